# baseline (speedup 1.0000x reference)
import functools

import jax
import jax.numpy as jnp
from jax import lax
from jax.experimental import pallas as pl
from jax.experimental.pallas import tpu as pltpu
from jax.experimental.pallas import tpu_sc as plsc

_L = 16
_K = 8
_ROWS = 64
_COLS = 8192
_CHUNKS = _COLS // _L
_GRP = 8
_NGRP = _CHUNKS // _GRP
_GELEM = _GRP * _L

_SORT8 = [(0, 1), (2, 3), (4, 5), (6, 7), (0, 2), (1, 3), (4, 6), (5, 7),
          (1, 2), (5, 6), (0, 4), (1, 5), (2, 6), (3, 7), (2, 4), (3, 5),
          (1, 2), (3, 4), (5, 6)]
_SORT4 = [(0, 1), (2, 3), (0, 2), (1, 3), (1, 2)]
_BITONIC8 = [(i, i + d) for d in (4, 2, 1) for i in range(8)
             if i & d == 0 and i + d < 8]


def _build():
    info = plsc.get_sparse_core_info()
    nc, ns = info.num_cores, info.num_subcores
    nw = nc * ns
    rows_per_w = _ROWS // nw
    rows_per_core = _ROWS // nc
    mesh = plsc.VectorSubcoreMesh(core_axis_name="c", subcore_axis_name="s")

    @functools.partial(
        pl.kernel,
        mesh=mesh,
        out_type=jax.ShapeDtypeStruct((_ROWS,), jnp.float32),
        scratch_types=[
            pltpu.VMEM((_COLS,), jnp.float32),
            pltpu.VMEM((_COLS,), jnp.float32),
            pltpu.VMEM((ns * _L,), jnp.float32),
            pltpu.VMEM((_L,), jnp.float32),
            pltpu.VMEM((2 * _L,), jnp.float32),
            pltpu.VMEM_SHARED((ns * _L,), jnp.float32),
            pltpu.SemaphoreType.DMA,
            pltpu.SemaphoreType.DMA,
        ],
    )
    def topk_mean(sim_hbm, out_hbm, rowa_v, rowb_v, buf_v, res_v, out_v,
                  shared_v, sema, semb):
        c = lax.axis_index("c")
        s = lax.axis_index("s")
        wid = c * ns + s
        ra = wid * rows_per_w
        npiece = 4
        plen = _COLS // npiece
        cpas = [pltpu.async_copy(sim_hbm.at[ra, pl.ds(plen * p, plen)],
                                 rowa_v.at[pl.ds(plen * p, plen)], sema)
                for p in range(npiece)]
        cpbs = [pltpu.async_copy(sim_hbm.at[ra + 1, pl.ds(plen * p, plen)],
                                 rowb_v.at[pl.ds(plen * p, plen)], semb)
                for p in range(npiece)]

        neg = jnp.full((_L,), -jnp.inf, dtype=jnp.float32)
        big = jnp.full((_L,), jnp.int32(2 * _NGRP), dtype=jnp.int32)
        lanes = lax.iota(jnp.int32, _L)
        perms = [jnp.bitwise_xor(lanes, d) for d in (1, 2, 4, 8)]
        dnums = lax.GatherDimensionNumbers(
            offset_dims=(), collapsed_slice_dims=(0,), start_index_map=(0,))

        def shuffle(x, p):
            return lax.gather(
                x, p[:, None], dnums, slice_sizes=(1,),
                mode=lax.GatherScatterMode.PROMISE_IN_BOUNDS)

        def xmax(x):
            for p in perms:
                x = jnp.maximum(x, shuffle(x, p))
            return x

        def xmin_i32(x):
            for p in perms:
                x = jnp.minimum(x, shuffle(x, p))
            return x

        def xsum_i32(x):
            for p in perms:
                x = x + shuffle(x, p)
            return x

        def cnet(v, pairs):
            v = list(v)
            for i, j in pairs:
                hi = jnp.maximum(v[i], v[j])
                lo = jnp.minimum(v[i], v[j])
                v[i], v[j] = hi, lo
            return v

        def lane_merge(stk):
            for p in perms:
                partner = [shuffle(x, p) for x in stk]
                m = [jnp.maximum(stk[i], partner[_K - 1 - i])
                     for i in range(_K)]
                stk = cnet(m, _BITONIC8)
            return stk

        def p1row(row_v):
            def body(u, carry):
                t = list(carry)
                lm = lanes == u
                for j in range(4):
                    base = (4 * u + j) * _GELEM
                    cs = [row_v[pl.ds(base + i * _L, _L)]
                          for i in range(_GRP)]
                    for step in (4, 2, 1):
                        for i in range(step):
                            cs[i] = jnp.maximum(cs[i], cs[i + step])
                    t[j] = jnp.where(lm, xmax(cs[0]), t[j])
                return tuple(t)
            return body

        tsa = (neg,) * 4
        for p in range(npiece):
            cpas[p].wait()
            tsa = lax.fori_loop(4 * p, 4 * p + 4, p1row(rowa_v), tsa)
        tsb = (neg,) * 4
        for p in range(npiece):
            cpbs[p].wait()
            tsb = lax.fori_loop(4 * p, 4 * p + 4, p1row(rowb_v), tsb)

        def thresh(Ts):
            stk = cnet(list(Ts), _SORT4) + [neg] * 4
            return lane_merge(stk)[_K - 1]

        t_a = thresh(tsa)
        t_b = thresh(tsb)

        def worklist(Ts, t):
            idxv = []
            nhit = jnp.zeros((_L,), jnp.int32)
            for j in range(4):
                hit = Ts[j] >= t
                idxv.append(jnp.where(hit, lanes * jnp.int32(4) + jnp.int32(j),
                                      big))
                nhit = nhit + jnp.where(hit, jnp.int32(1), jnp.int32(0))
            return idxv, xsum_i32(nhit)[0]

        idxa, cnt_a = worklist(tsa, t_a)
        idxb, cnt_b = worklist(tsb, t_b)

        def make_insert(row_v):
            def insert_body(k, carry):
                st = list(carry[:_K])
                iv = list(carry[_K:])
                cand = jnp.minimum(jnp.minimum(iv[0], iv[1]),
                                   jnp.minimum(iv[2], iv[3]))
                g = xmin_i32(cand)
                for j in range(4):
                    iv[j] = jnp.where(iv[j] == g, big, iv[j])
                base = g[0] * jnp.int32(_GELEM)
                cs = [row_v[pl.ds(base + i * _L, _L)] for i in range(_GRP)]
                grp = cnet(cs, _SORT8)
                m = [jnp.maximum(st[i], grp[_K - 1 - i]) for i in range(_K)]
                return tuple(cnet(m, _BITONIC8)) + tuple(iv)

            return insert_body

        carry_a = lax.fori_loop(0, cnt_a, make_insert(rowa_v),
                                (neg,) * _K + tuple(idxa))
        carry_b = lax.fori_loop(0, cnt_b, make_insert(rowb_v),
                                (neg,) * _K + tuple(idxb))

        def mean_of(stack):
            g = lane_merge(list(stack))
            tot = g[0]
            for i in range(1, _K):
                tot = tot + g[i]
            return tot * jnp.float32(1.0 / _K)

        mean_a = mean_of(carry_a[:_K])
        mean_b = mean_of(carry_b[:_K])

        res_v[...] = jnp.where(lanes == 0, mean_a, mean_b)
        pltpu.sync_copy(res_v, shared_v.at[pl.ds(s * _L, _L)])
        plsc.subcore_barrier()

        def write_out():
            pltpu.sync_copy(shared_v, buf_v)
            rows = [buf_v[pl.ds(i * _L, _L)] for i in range(ns)]
            p0 = jnp.zeros((_L,), jnp.int32)
            p1 = p0 + jnp.int32(1)
            out0 = neg
            out1 = neg
            for i in range(_K):
                m0 = shuffle(rows[i], p0)
                m1 = shuffle(rows[i], p1)
                out0 = jnp.where(lanes == 2 * i, m0, out0)
                out0 = jnp.where(lanes == 2 * i + 1, m1, out0)
            for i in range(_K, ns):
                m0 = shuffle(rows[i], p0)
                m1 = shuffle(rows[i], p1)
                out1 = jnp.where(lanes == 2 * i - _L, m0, out1)
                out1 = jnp.where(lanes == 2 * i + 1 - _L, m1, out1)
            out_v[pl.ds(0, _L)] = out0
            out_v[pl.ds(_L, _L)] = out1
            pltpu.sync_copy(out_v,
                            out_hbm.at[pl.ds(c * rows_per_core, 2 * _L)])

        lax.cond(s == 0, write_out, lambda: None)

    return topk_mean


_topk_mean = _build()


@jax.jit
def kernel(sim):
    return _topk_mean(sim)


# speedup vs baseline: 1.0356x; 1.0356x over previous
import functools

import jax
import jax.numpy as jnp
from jax import lax
from jax.experimental import pallas as pl
from jax.experimental.pallas import tpu as pltpu
from jax.experimental.pallas import tpu_sc as plsc

_L = 16
_K = 8
_ROWS = 64
_COLS = 8192
_CHUNKS = _COLS // _L
_GRP = 8
_NGRP = _CHUNKS // _GRP
_GELEM = _GRP * _L

_SORT8 = [(0, 1), (2, 3), (4, 5), (6, 7), (0, 2), (1, 3), (4, 6), (5, 7),
          (1, 2), (5, 6), (0, 4), (1, 5), (2, 6), (3, 7), (2, 4), (3, 5),
          (1, 2), (3, 4), (5, 6)]
_SORT4 = [(0, 1), (2, 3), (0, 2), (1, 3), (1, 2)]
_BITONIC8 = [(i, i + d) for d in (4, 2, 1) for i in range(8)
             if i & d == 0 and i + d < 8]


def _build():
    info = plsc.get_sparse_core_info()
    nc, ns = info.num_cores, info.num_subcores
    nw = nc * ns
    rows_per_w = _ROWS // nw
    rows_per_core = _ROWS // nc
    mesh = plsc.VectorSubcoreMesh(core_axis_name="c", subcore_axis_name="s")

    @functools.partial(
        pl.kernel,
        mesh=mesh,
        out_type=jax.ShapeDtypeStruct((_ROWS,), jnp.float32),
        scratch_types=[
            pltpu.VMEM((_COLS,), jnp.float32),
            pltpu.VMEM((_COLS,), jnp.float32),
            pltpu.VMEM((ns * _L,), jnp.float32),
            pltpu.VMEM((_L,), jnp.float32),
            pltpu.VMEM((2 * _L,), jnp.float32),
            pltpu.VMEM_SHARED((ns * _L,), jnp.float32),
            pltpu.SemaphoreType.DMA,
            pltpu.SemaphoreType.DMA,
        ],
    )
    def topk_mean(sim_hbm, out_hbm, rowa_v, rowb_v, buf_v, res_v, out_v,
                  shared_v, sema, semb):
        c = lax.axis_index("c")
        s = lax.axis_index("s")
        wid = c * ns + s
        ra = wid * rows_per_w
        npiece = 2
        plen = _COLS // npiece
        cpas = [pltpu.async_copy(sim_hbm.at[ra, pl.ds(plen * p, plen)],
                                 rowa_v.at[pl.ds(plen * p, plen)], sema)
                for p in range(npiece)]
        cpbs = [pltpu.async_copy(sim_hbm.at[ra + 1, pl.ds(plen * p, plen)],
                                 rowb_v.at[pl.ds(plen * p, plen)], semb)
                for p in range(npiece)]

        neg = jnp.full((_L,), -jnp.inf, dtype=jnp.float32)
        big = jnp.full((_L,), jnp.int32(2 * _NGRP), dtype=jnp.int32)
        lanes = lax.iota(jnp.int32, _L)
        perms = [jnp.bitwise_xor(lanes, d) for d in (1, 2, 4, 8)]
        dnums = lax.GatherDimensionNumbers(
            offset_dims=(), collapsed_slice_dims=(0,), start_index_map=(0,))

        def shuffle(x, p):
            return lax.gather(
                x, p[:, None], dnums, slice_sizes=(1,),
                mode=lax.GatherScatterMode.PROMISE_IN_BOUNDS)

        def xmax(x):
            for p in perms:
                x = jnp.maximum(x, shuffle(x, p))
            return x

        def xmin_i32(x):
            for p in perms:
                x = jnp.minimum(x, shuffle(x, p))
            return x

        def xsum_i32(x):
            for p in perms:
                x = x + shuffle(x, p)
            return x

        def cnet(v, pairs):
            v = list(v)
            for i, j in pairs:
                hi = jnp.maximum(v[i], v[j])
                lo = jnp.minimum(v[i], v[j])
                v[i], v[j] = hi, lo
            return v

        def lane_merge(stk):
            for p in perms:
                partner = [shuffle(x, p) for x in stk]
                m = [jnp.maximum(stk[i], partner[_K - 1 - i])
                     for i in range(_K)]
                stk = cnet(m, _BITONIC8)
            return stk

        def p1row(row_v):
            def body(u, carry):
                t = list(carry)
                lm = lanes == u
                for j in range(4):
                    base = (4 * u + j) * _GELEM
                    cs = [row_v[pl.ds(base + i * _L, _L)]
                          for i in range(_GRP)]
                    for step in (4, 2, 1):
                        for i in range(step):
                            cs[i] = jnp.maximum(cs[i], cs[i + step])
                    t[j] = jnp.where(lm, xmax(cs[0]), t[j])
                return tuple(t)
            return body

        upp = _L // npiece
        tsa = (neg,) * 4
        for p in range(npiece):
            cpas[p].wait()
            tsa = lax.fori_loop(upp * p, upp * p + upp, p1row(rowa_v), tsa)
        tsb = (neg,) * 4
        for p in range(npiece):
            cpbs[p].wait()
            tsb = lax.fori_loop(upp * p, upp * p + upp, p1row(rowb_v), tsb)

        def thresh(Ts):
            stk = cnet(list(Ts), _SORT4) + [neg] * 4
            return lane_merge(stk)[_K - 1]

        t_a = thresh(tsa)
        t_b = thresh(tsb)

        def worklist(Ts, t):
            idxv = []
            nhit = jnp.zeros((_L,), jnp.int32)
            for j in range(4):
                hit = Ts[j] >= t
                idxv.append(jnp.where(hit, lanes * jnp.int32(4) + jnp.int32(j),
                                      big))
                nhit = nhit + jnp.where(hit, jnp.int32(1), jnp.int32(0))
            return idxv, xsum_i32(nhit)[0]

        idxa, cnt_a = worklist(tsa, t_a)
        idxb, cnt_b = worklist(tsb, t_b)

        def make_insert(row_v):
            def insert_body(k, carry):
                st = list(carry[:_K])
                iv = list(carry[_K:])
                cand = jnp.minimum(jnp.minimum(iv[0], iv[1]),
                                   jnp.minimum(iv[2], iv[3]))
                g = xmin_i32(cand)
                for j in range(4):
                    iv[j] = jnp.where(iv[j] == g, big, iv[j])
                base = g[0] * jnp.int32(_GELEM)
                cs = [row_v[pl.ds(base + i * _L, _L)] for i in range(_GRP)]
                grp = cnet(cs, _SORT8)
                m = [jnp.maximum(st[i], grp[_K - 1 - i]) for i in range(_K)]
                return tuple(cnet(m, _BITONIC8)) + tuple(iv)

            return insert_body

        carry_a = lax.fori_loop(0, cnt_a, make_insert(rowa_v),
                                (neg,) * _K + tuple(idxa))
        carry_b = lax.fori_loop(0, cnt_b, make_insert(rowb_v),
                                (neg,) * _K + tuple(idxb))

        def mean_of(stack):
            g = lane_merge(list(stack))
            tot = g[0]
            for i in range(1, _K):
                tot = tot + g[i]
            return tot * jnp.float32(1.0 / _K)

        mean_a = mean_of(carry_a[:_K])
        mean_b = mean_of(carry_b[:_K])

        res_v[...] = jnp.where(lanes == 0, mean_a, mean_b)
        pltpu.sync_copy(res_v, shared_v.at[pl.ds(s * _L, _L)])
        plsc.subcore_barrier()

        def write_out():
            pltpu.sync_copy(shared_v, buf_v)
            rows = [buf_v[pl.ds(i * _L, _L)] for i in range(ns)]
            p0 = jnp.zeros((_L,), jnp.int32)
            p1 = p0 + jnp.int32(1)
            out0 = neg
            out1 = neg
            for i in range(_K):
                m0 = shuffle(rows[i], p0)
                m1 = shuffle(rows[i], p1)
                out0 = jnp.where(lanes == 2 * i, m0, out0)
                out0 = jnp.where(lanes == 2 * i + 1, m1, out0)
            for i in range(_K, ns):
                m0 = shuffle(rows[i], p0)
                m1 = shuffle(rows[i], p1)
                out1 = jnp.where(lanes == 2 * i - _L, m0, out1)
                out1 = jnp.where(lanes == 2 * i + 1 - _L, m1, out1)
            out_v[pl.ds(0, _L)] = out0
            out_v[pl.ds(_L, _L)] = out1
            pltpu.sync_copy(out_v,
                            out_hbm.at[pl.ds(c * rows_per_core, 2 * _L)])

        lax.cond(s == 0, write_out, lambda: None)

    return topk_mean


_topk_mean = _build()


@jax.jit
def kernel(sim):
    return _topk_mean(sim)


# speedup vs baseline: 1.0721x; 1.0352x over previous
import functools

import jax
import jax.numpy as jnp
from jax import lax
from jax.experimental import pallas as pl
from jax.experimental.pallas import tpu as pltpu
from jax.experimental.pallas import tpu_sc as plsc

_L = 16
_K = 8
_ROWS = 64
_COLS = 8192
_CHUNKS = _COLS // _L
_GRP = 8
_NGRP = _CHUNKS // _GRP
_GELEM = _GRP * _L

_SORT8 = [(0, 1), (2, 3), (4, 5), (6, 7), (0, 2), (1, 3), (4, 6), (5, 7),
          (1, 2), (5, 6), (0, 4), (1, 5), (2, 6), (3, 7), (2, 4), (3, 5),
          (1, 2), (3, 4), (5, 6)]
_SORT4 = [(0, 1), (2, 3), (0, 2), (1, 3), (1, 2)]
_BITONIC8 = [(i, i + d) for d in (4, 2, 1) for i in range(8)
             if i & d == 0 and i + d < 8]


def _build():
    info = plsc.get_sparse_core_info()
    nc, ns = info.num_cores, info.num_subcores
    nw = nc * ns
    rows_per_w = _ROWS // nw
    rows_per_core = _ROWS // nc
    mesh = plsc.VectorSubcoreMesh(core_axis_name="c", subcore_axis_name="s")

    @functools.partial(
        pl.kernel,
        mesh=mesh,
        out_type=jax.ShapeDtypeStruct((_ROWS,), jnp.float32),
        scratch_types=[
            pltpu.VMEM((_COLS,), jnp.float32),
            pltpu.VMEM((_COLS,), jnp.float32),
            pltpu.VMEM((ns * _L,), jnp.float32),
            pltpu.VMEM((_L,), jnp.float32),
            pltpu.VMEM((2 * _L,), jnp.float32),
            pltpu.VMEM_SHARED((ns * _L,), jnp.float32),
            pltpu.SemaphoreType.DMA,
            pltpu.SemaphoreType.DMA,
        ],
    )
    def topk_mean(sim_hbm, out_hbm, rowa_v, rowb_v, buf_v, res_v, out_v,
                  shared_v, sema, semb):
        c = lax.axis_index("c")
        s = lax.axis_index("s")
        wid = c * ns + s
        ra = wid * rows_per_w
        cpa = pltpu.async_copy(sim_hbm.at[ra], rowa_v, sema)
        cpb = pltpu.async_copy(sim_hbm.at[ra + 1], rowb_v, semb)

        neg = jnp.full((_L,), -jnp.inf, dtype=jnp.float32)
        big = jnp.full((_L,), jnp.int32(2 * _NGRP), dtype=jnp.int32)
        lanes = lax.iota(jnp.int32, _L)
        perms = [jnp.bitwise_xor(lanes, d) for d in (1, 2, 4, 8)]
        dnums = lax.GatherDimensionNumbers(
            offset_dims=(), collapsed_slice_dims=(0,), start_index_map=(0,))

        def shuffle(x, p):
            return lax.gather(
                x, p[:, None], dnums, slice_sizes=(1,),
                mode=lax.GatherScatterMode.PROMISE_IN_BOUNDS)

        def xmax(x):
            for p in perms:
                x = jnp.maximum(x, shuffle(x, p))
            return x

        def xmin_i32(x):
            for p in perms:
                x = jnp.minimum(x, shuffle(x, p))
            return x

        def xsum_i32(x):
            for p in perms:
                x = x + shuffle(x, p)
            return x

        def cnet(v, pairs):
            v = list(v)
            for i, j in pairs:
                hi = jnp.maximum(v[i], v[j])
                lo = jnp.minimum(v[i], v[j])
                v[i], v[j] = hi, lo
            return v

        def lane_merge(stk):
            for p in perms:
                partner = [shuffle(x, p) for x in stk]
                m = [jnp.maximum(stk[i], partner[_K - 1 - i])
                     for i in range(_K)]
                stk = cnet(m, _BITONIC8)
            return stk

        cpa.wait()
        cpb.wait()

        def p1body(u, carry):
            ta, tb = list(carry[:4]), list(carry[4:])
            lm = lanes == u
            for j in range(4):
                base = (4 * u + j) * _GELEM
                ca = [rowa_v[pl.ds(base + i * _L, _L)] for i in range(_GRP)]
                cb = [rowb_v[pl.ds(base + i * _L, _L)] for i in range(_GRP)]
                for step in (4, 2, 1):
                    for i in range(step):
                        ca[i] = jnp.maximum(ca[i], ca[i + step])
                        cb[i] = jnp.maximum(cb[i], cb[i + step])
                ta[j] = jnp.where(lm, xmax(ca[0]), ta[j])
                tb[j] = jnp.where(lm, xmax(cb[0]), tb[j])
            return tuple(ta) + tuple(tb)

        tcarry = lax.fori_loop(0, _L, p1body, (neg,) * 8)
        tsa, tsb = tcarry[:4], tcarry[4:]

        def thresh(Ts):
            stk = cnet(list(Ts), _SORT4) + [neg] * 4
            return lane_merge(stk)[_K - 1]

        t_a = thresh(tsa)
        t_b = thresh(tsb)

        def worklist(Ts, t):
            idxv = []
            nhit = jnp.zeros((_L,), jnp.int32)
            for j in range(4):
                hit = Ts[j] >= t
                idxv.append(jnp.where(hit, lanes * jnp.int32(4) + jnp.int32(j),
                                      big))
                nhit = nhit + jnp.where(hit, jnp.int32(1), jnp.int32(0))
            return idxv, xsum_i32(nhit)[0]

        idxa, cnt_a = worklist(tsa, t_a)
        idxb, cnt_b = worklist(tsb, t_b)

        def make_insert(row_v):
            def insert_body(k, carry):
                st = list(carry[:_K])
                iv = list(carry[_K:])
                cand = jnp.minimum(jnp.minimum(iv[0], iv[1]),
                                   jnp.minimum(iv[2], iv[3]))
                g = xmin_i32(cand)
                for j in range(4):
                    iv[j] = jnp.where(iv[j] == g, big, iv[j])
                base = g[0] * jnp.int32(_GELEM)
                cs = [row_v[pl.ds(base + i * _L, _L)] for i in range(_GRP)]
                grp = cnet(cs, _SORT8)
                m = [jnp.maximum(st[i], grp[_K - 1 - i]) for i in range(_K)]
                return tuple(cnet(m, _BITONIC8)) + tuple(iv)

            return insert_body

        carry_a = lax.fori_loop(0, cnt_a, make_insert(rowa_v),
                                (neg,) * _K + tuple(idxa))
        carry_b = lax.fori_loop(0, cnt_b, make_insert(rowb_v),
                                (neg,) * _K + tuple(idxb))

        def mean_of(stack):
            g = lane_merge(list(stack))
            tot = g[0]
            for i in range(1, _K):
                tot = tot + g[i]
            return tot * jnp.float32(1.0 / _K)

        mean_a = mean_of(carry_a[:_K])
        mean_b = mean_of(carry_b[:_K])

        res_v[...] = jnp.where(lanes == 0, mean_a, mean_b)
        pltpu.sync_copy(res_v, shared_v.at[pl.ds(s * _L, _L)])
        plsc.subcore_barrier()

        def write_out():
            pltpu.sync_copy(shared_v, buf_v)
            rows = [buf_v[pl.ds(i * _L, _L)] for i in range(ns)]
            p0 = jnp.zeros((_L,), jnp.int32)
            p1 = p0 + jnp.int32(1)
            out0 = neg
            out1 = neg
            for i in range(_K):
                m0 = shuffle(rows[i], p0)
                m1 = shuffle(rows[i], p1)
                out0 = jnp.where(lanes == 2 * i, m0, out0)
                out0 = jnp.where(lanes == 2 * i + 1, m1, out0)
            for i in range(_K, ns):
                m0 = shuffle(rows[i], p0)
                m1 = shuffle(rows[i], p1)
                out1 = jnp.where(lanes == 2 * i - _L, m0, out1)
                out1 = jnp.where(lanes == 2 * i + 1 - _L, m1, out1)
            out_v[pl.ds(0, _L)] = out0
            out_v[pl.ds(_L, _L)] = out1
            pltpu.sync_copy(out_v,
                            out_hbm.at[pl.ds(c * rows_per_core, 2 * _L)])

        lax.cond(s == 0, write_out, lambda: None)

    return topk_mean


_topk_mean = _build()


@jax.jit
def kernel(sim):
    return _topk_mean(sim)
